# baseline (speedup 1.0000x reference)
import jax
import jax.numpy as jnp
from jax.experimental import pallas as pl
from jax.experimental.pallas import tpu as pltpu

_N_E = 1024
_E_DIM = 64
_BETA = 0.25
_B = 16
_HW = 1024
_N_TOTAL = _B * _HW * _E_DIM
_R = 1024
_NSTEP = _B * _HW // _R


def _vq_body(z_ref, e_ref, zq_ref, enc_ref, idx_ref, loss_ref, perp_ref,
             hist_ref, eye_ref):
    i = pl.program_id(0)

    @pl.when(i == 0)
    def _eye():
        eye_ref[...] = (
            jax.lax.broadcasted_iota(jnp.int32, (_R, _R), 0)
            == jax.lax.broadcasted_iota(jnp.int32, (_R, _R), 1)
        ).astype(jnp.float32)
    zt = z_ref[...]
    ewT = e_ref[...]
    ew = ewT.T

    esq = jnp.sum(ew * ew, axis=1)
    zsq = jnp.sum(zt * zt, axis=1, keepdims=True)
    mmn = jax.lax.dot_general(-2.0 * zt, ewT, (((1,), (0,)), ((), ())),
                              preferred_element_type=jnp.float32)
    d = (zsq + esq) + mmn

    dmin = jnp.min(d, axis=1, keepdims=True)
    iota = jax.lax.broadcasted_iota(jnp.int32, d.shape, 1).astype(jnp.float32)
    keys = jnp.where(d == dmin, iota, jnp.float32(2.0 * _N_E))
    idx_f = jnp.min(keys, axis=1, keepdims=True)
    one_hot = (keys == idx_f).astype(jnp.float32)

    enc_ref[...] = one_hot
    idx_row = jax.lax.dot_general(idx_f, eye_ref[...], (((0,), (0,)), ((), ())),
                                  precision=jax.lax.Precision.HIGHEST,
                                  preferred_element_type=jnp.float32)
    idx_ref[...] = idx_row.reshape(_R // 128, 128).astype(jnp.int32)

    zq_ref[...] = jax.lax.dot_general(one_hot, ewT, (((1,), (1,)), ((), ())),
                                      preferred_element_type=jnp.float32)

    part = jnp.sum(dmin)
    hpart = jnp.sum(one_hot, axis=0)[None, :]

    @pl.when(i == 0)
    def _init():
        loss_ref[...] = jnp.zeros_like(loss_ref)
        hist_ref[...] = jnp.zeros_like(hist_ref)

    loss_ref[...] = loss_ref[...] + part
    hist_ref[...] = hist_ref[...] + hpart

    @pl.when(i == pl.num_programs(0) - 1)
    def _finish():
        loss_ref[...] = (1.0 + _BETA) * loss_ref[...] / _N_TOTAL
        e_mean = hist_ref[...] / (_B * _HW)
        ent = jnp.sum(e_mean * jnp.log(e_mean + 1e-10))
        perp_ref[...] = jnp.exp(-ent) * jnp.ones_like(perp_ref)


def kernel(z, emb_weight):
    zf = jnp.transpose(z, (0, 2, 3, 1)).reshape(_B * _HW, _E_DIM)
    et = emb_weight.T
    zq2, enc, idx, loss, perp = pl.pallas_call(
        _vq_body,
        grid=(_NSTEP,),
        in_specs=[
            pl.BlockSpec((_R, _E_DIM), lambda i: (i, 0)),
            pl.BlockSpec((_E_DIM, _N_E), lambda i: (0, 0)),
        ],
        out_specs=[
            pl.BlockSpec((_R, _E_DIM), lambda i: (i, 0)),
            pl.BlockSpec((_R, _N_E), lambda i: (i, 0)),
            pl.BlockSpec((_R // 128, 128), lambda i: (i, 0)),
            pl.BlockSpec((1, 1), lambda i: (0, 0)),
            pl.BlockSpec((1, 1), lambda i: (0, 0)),
        ],
        out_shape=[
            jax.ShapeDtypeStruct((_B * _HW, _E_DIM), jnp.float32),
            jax.ShapeDtypeStruct((_B * _HW, _N_E), jnp.float32),
            jax.ShapeDtypeStruct((_B * _HW // 128, 128), jnp.int32),
            jax.ShapeDtypeStruct((1, 1), jnp.float32),
            jax.ShapeDtypeStruct((1, 1), jnp.float32),
        ],
        scratch_shapes=[pltpu.VMEM((1, _N_E), jnp.float32),
                        pltpu.VMEM((_R, _R), jnp.float32)],
    )(zf, et)
    zq = jnp.transpose(zq2.reshape(_B, 32, 32, _E_DIM), (0, 3, 1, 2))
    return (zq, loss.reshape(()), perp.reshape(()), enc,
            idx.reshape(_B * _HW, 1))


# speedup vs baseline: 1.2153x; 1.2153x over previous
import jax
import jax.numpy as jnp
from jax.experimental import pallas as pl
from jax.experimental.pallas import tpu as pltpu

_N_E = 1024
_E_DIM = 64
_BETA = 0.25
_B = 16
_HW = 1024
_N_TOTAL = _B * _HW * _E_DIM
_R = 1024
_NSTEP = _B * _HW // _R


def _vq_body(z_ref, e_ref, zq_ref, enc_ref, idx_ref, loss_ref, perp_ref,
             hist_ref):
    i = pl.program_id(0)
    zt = z_ref[...]
    ewT = e_ref[...]
    ew = ewT.T

    esq = jnp.sum(ew * ew, axis=1)
    zsq = jnp.sum(zt * zt, axis=1, keepdims=True)
    mmn = jax.lax.dot_general(-2.0 * zt, ew, (((1,), (1,)), ((), ())),
                              preferred_element_type=jnp.float32)
    d = (zsq + esq) + mmn

    dmin = jnp.min(d, axis=1, keepdims=True)
    iota = jax.lax.broadcasted_iota(jnp.int32, d.shape, 1).astype(jnp.float32)
    keys = jnp.where(d == dmin, iota, jnp.float32(2.0 * _N_E))
    idx_f = jnp.min(keys, axis=1, keepdims=True)
    one_hot = (keys == idx_f).astype(jnp.float32)

    enc_ref[...] = one_hot
    idx_ref[...] = idx_f.astype(jnp.int32)

    zq_ref[...] = jax.lax.dot_general(one_hot, ew, (((1,), (0,)), ((), ())),
                                      preferred_element_type=jnp.float32)

    part = jnp.sum(dmin)
    hpart = jnp.sum(one_hot, axis=0)[None, :]

    @pl.when(i == 0)
    def _init():
        loss_ref[...] = jnp.zeros_like(loss_ref)
        hist_ref[...] = jnp.zeros_like(hist_ref)

    loss_ref[...] = loss_ref[...] + part
    hist_ref[...] = hist_ref[...] + hpart

    @pl.when(i == pl.num_programs(0) - 1)
    def _finish():
        loss_ref[...] = (1.0 + _BETA) * loss_ref[...] / _N_TOTAL
        e_mean = hist_ref[...] / (_B * _HW)
        ent = jnp.sum(e_mean * jnp.log(e_mean + 1e-10))
        perp_ref[...] = jnp.exp(-ent) * jnp.ones_like(perp_ref)


def kernel(z, emb_weight):
    zf = jnp.transpose(z, (0, 2, 3, 1)).reshape(_B * _HW, _E_DIM)
    et = emb_weight.T
    zq2, enc, idx, loss, perp = pl.pallas_call(
        _vq_body,
        grid=(_NSTEP,),
        in_specs=[
            pl.BlockSpec((_R, _E_DIM), lambda i: (i, 0)),
            pl.BlockSpec((_E_DIM, _N_E), lambda i: (0, 0)),
        ],
        out_specs=[
            pl.BlockSpec((_R, _E_DIM), lambda i: (i, 0)),
            pl.BlockSpec((_R, _N_E), lambda i: (i, 0)),
            pl.BlockSpec((_R, 1), lambda i: (i, 0)),
            pl.BlockSpec((1, 1), lambda i: (0, 0)),
            pl.BlockSpec((1, 1), lambda i: (0, 0)),
        ],
        out_shape=[
            jax.ShapeDtypeStruct((_B * _HW, _E_DIM), jnp.float32),
            jax.ShapeDtypeStruct((_B * _HW, _N_E), jnp.float32),
            jax.ShapeDtypeStruct((_B * _HW, 1), jnp.int32),
            jax.ShapeDtypeStruct((1, 1), jnp.float32),
            jax.ShapeDtypeStruct((1, 1), jnp.float32),
        ],
        scratch_shapes=[pltpu.VMEM((1, _N_E), jnp.float32)],
    )(zf, et)
    zq = jnp.transpose(zq2.reshape(_B, 32, 32, _E_DIM), (0, 3, 1, 2))
    return (zq, loss.reshape(()), perp.reshape(()), enc, idx)


# speedup vs baseline: 1.3967x; 1.1493x over previous
import jax
import jax.numpy as jnp
from jax.experimental import pallas as pl
from jax.experimental.pallas import tpu as pltpu

_N_E = 1024
_E_DIM = 64
_BETA = 0.25
_B = 16
_HW = 1024
_N_TOTAL = _B * _HW * _E_DIM
_R = 1024
_NSTEP = _B * _HW // _R


def _vq_body(z_ref, e_ref, et_ref, zq_ref, enc_ref, idx_ref, loss_ref,
             perp_ref, hist_ref):
    i = pl.program_id(0)
    zt = z_ref[...]
    ew = e_ref[...]
    ewT = et_ref[...]

    esq = jnp.sum(ew * ew, axis=1)
    zsq = jnp.sum(zt * zt, axis=1, keepdims=True)
    mmn = jax.lax.dot_general(-2.0 * zt, ewT, (((1,), (0,)), ((), ())),
                              preferred_element_type=jnp.float32)
    d = (zsq + esq) + mmn

    dmin = jnp.min(d, axis=1, keepdims=True)
    iota = jax.lax.broadcasted_iota(jnp.int32, d.shape, 1).astype(jnp.float32)
    keys = jnp.where(d == dmin, iota, jnp.float32(2.0 * _N_E))
    idx_f = jnp.min(keys, axis=1, keepdims=True)
    one_hot = (keys == idx_f).astype(jnp.float32)

    enc_ref[...] = one_hot
    idx_ref[...] = idx_f.astype(jnp.int32)

    zq_ref[...] = jax.lax.dot_general(one_hot, ew, (((1,), (0,)), ((), ())),
                                      preferred_element_type=jnp.float32)

    part = jnp.sum(dmin)
    hpart = jnp.sum(one_hot, axis=0)[None, :]

    @pl.when(i == 0)
    def _init():
        loss_ref[...] = jnp.zeros_like(loss_ref)
        hist_ref[...] = jnp.zeros_like(hist_ref)

    loss_ref[...] = loss_ref[...] + part
    hist_ref[...] = hist_ref[...] + hpart

    @pl.when(i == pl.num_programs(0) - 1)
    def _finish():
        loss_ref[...] = (1.0 + _BETA) * loss_ref[...] / _N_TOTAL
        e_mean = hist_ref[...] / (_B * _HW)
        ent = jnp.sum(e_mean * jnp.log(e_mean + 1e-10))
        perp_ref[...] = jnp.exp(-ent) * jnp.ones_like(perp_ref)


def kernel(z, emb_weight):
    zf = jnp.transpose(z, (0, 2, 3, 1)).reshape(_B * _HW, _E_DIM)
    et = emb_weight.T
    zq2, enc, idx, loss, perp = pl.pallas_call(
        _vq_body,
        grid=(_NSTEP,),
        in_specs=[
            pl.BlockSpec((_R, _E_DIM), lambda i: (i, 0)),
            pl.BlockSpec((_N_E, _E_DIM), lambda i: (0, 0)),
            pl.BlockSpec((_E_DIM, _N_E), lambda i: (0, 0)),
        ],
        out_specs=[
            pl.BlockSpec((_R, _E_DIM), lambda i: (i, 0)),
            pl.BlockSpec((_R, _N_E), lambda i: (i, 0)),
            pl.BlockSpec((_R, 1), lambda i: (i, 0)),
            pl.BlockSpec((1, 1), lambda i: (0, 0)),
            pl.BlockSpec((1, 1), lambda i: (0, 0)),
        ],
        out_shape=[
            jax.ShapeDtypeStruct((_B * _HW, _E_DIM), jnp.float32),
            jax.ShapeDtypeStruct((_B * _HW, _N_E), jnp.float32),
            jax.ShapeDtypeStruct((_B * _HW, 1), jnp.int32),
            jax.ShapeDtypeStruct((1, 1), jnp.float32),
            jax.ShapeDtypeStruct((1, 1), jnp.float32),
        ],
        scratch_shapes=[pltpu.VMEM((1, _N_E), jnp.float32)],
    )(zf, emb_weight, et)
    zq = jnp.transpose(zq2.reshape(_B, 32, 32, _E_DIM), (0, 3, 1, 2))
    return (zq, loss.reshape(()), perp.reshape(()), enc, idx)


# speedup vs baseline: 1.4717x; 1.0537x over previous
import jax
import jax.numpy as jnp
from jax.experimental import pallas as pl
from jax.experimental.pallas import tpu as pltpu

_N_E = 1024
_E_DIM = 64
_BETA = 0.25
_B = 16
_HW = 1024
_N_TOTAL = _B * _HW * _E_DIM
_R = 1024
_NSTEP = _B * _HW // _R


def _vq_body(z_ref, e_ref, zq_ref, enc_ref, idx_ref, loss_ref, perp_ref,
             hist_ref):
    i = pl.program_id(0)
    zt = z_ref[...]
    ewT = e_ref[...]
    ew = ewT.T

    esq = jnp.sum(ew * ew, axis=1)
    zsq = jnp.sum(zt * zt, axis=1, keepdims=True)
    mmn = jax.lax.dot_general(-2.0 * zt, ewT, (((1,), (0,)), ((), ())),
                              preferred_element_type=jnp.float32)
    d = (zsq + esq) + mmn

    dmin = jnp.min(d, axis=1, keepdims=True)
    iota = jax.lax.broadcasted_iota(jnp.int32, d.shape, 1).astype(jnp.float32)
    keys = jnp.where(d == dmin, iota, jnp.float32(2.0 * _N_E))
    idx_f = jnp.min(keys, axis=1, keepdims=True)
    one_hot = (keys == idx_f).astype(jnp.float32)

    enc_ref[...] = one_hot
    idx_ref[...] = idx_f.astype(jnp.int32)

    zq_ref[...] = jax.lax.dot_general(one_hot, ewT, (((1,), (1,)), ((), ())),
                                      preferred_element_type=jnp.float32)

    part = jnp.sum(dmin)
    hpart = jnp.sum(one_hot, axis=0)[None, :]

    @pl.when(i == 0)
    def _init():
        loss_ref[...] = jnp.zeros_like(loss_ref)
        hist_ref[...] = jnp.zeros_like(hist_ref)

    loss_ref[...] = loss_ref[...] + part
    hist_ref[...] = hist_ref[...] + hpart

    @pl.when(i == pl.num_programs(0) - 1)
    def _finish():
        loss_ref[...] = (1.0 + _BETA) * loss_ref[...] / _N_TOTAL
        e_mean = hist_ref[...] / (_B * _HW)
        ent = jnp.sum(e_mean * jnp.log(e_mean + 1e-10))
        perp_ref[...] = jnp.exp(-ent) * jnp.ones_like(perp_ref)


def kernel(z, emb_weight):
    zf = jnp.transpose(z, (0, 2, 3, 1)).reshape(_B * _HW, _E_DIM)
    et = emb_weight.T
    zq2, enc, idx, loss, perp = pl.pallas_call(
        _vq_body,
        grid=(_NSTEP,),
        in_specs=[
            pl.BlockSpec((_R, _E_DIM), lambda i: (i, 0)),
            pl.BlockSpec((_E_DIM, _N_E), lambda i: (0, 0)),
        ],
        out_specs=[
            pl.BlockSpec((_R, _E_DIM), lambda i: (i, 0)),
            pl.BlockSpec((_R, _N_E), lambda i: (i, 0)),
            pl.BlockSpec((_R, 1), lambda i: (i, 0)),
            pl.BlockSpec((1, 1), lambda i: (0, 0)),
            pl.BlockSpec((1, 1), lambda i: (0, 0)),
        ],
        out_shape=[
            jax.ShapeDtypeStruct((_B * _HW, _E_DIM), jnp.float32),
            jax.ShapeDtypeStruct((_B * _HW, _N_E), jnp.float32),
            jax.ShapeDtypeStruct((_B * _HW, 1), jnp.int32),
            jax.ShapeDtypeStruct((1, 1), jnp.float32),
            jax.ShapeDtypeStruct((1, 1), jnp.float32),
        ],
        scratch_shapes=[pltpu.VMEM((1, _N_E), jnp.float32)],
    )(zf, et)
    zq = jnp.transpose(zq2.reshape(_B, 32, 32, _E_DIM), (0, 3, 1, 2))
    return (zq, loss.reshape(()), perp.reshape(()), enc, idx)


# speedup vs baseline: 1.4866x; 1.0101x over previous
import jax
import jax.numpy as jnp
from jax.experimental import pallas as pl
from jax.experimental.pallas import tpu as pltpu

_N_E = 1024
_E_DIM = 64
_BETA = 0.25
_B = 16
_HW = 1024
_N_TOTAL = _B * _HW * _E_DIM
_R = 2048
_NSTEP = _B * _HW // _R


def _vq_body(z_ref, e_ref, zq_ref, enc_ref, idx_ref, loss_ref, perp_ref,
             hist_ref):
    i = pl.program_id(0)
    zt = z_ref[...]
    ewT = e_ref[...]
    ew = ewT.T

    esq = jnp.sum(ew * ew, axis=1)
    zsq = jnp.sum(zt * zt, axis=1, keepdims=True)
    mmn = jax.lax.dot_general(-2.0 * zt, ewT, (((1,), (0,)), ((), ())),
                              preferred_element_type=jnp.float32)
    d = (zsq + esq) + mmn

    dmin = jnp.min(d, axis=1, keepdims=True)
    iota = jax.lax.broadcasted_iota(jnp.int32, d.shape, 1).astype(jnp.float32)
    keys = jnp.where(d == dmin, iota, jnp.float32(2.0 * _N_E))
    idx_f = jnp.min(keys, axis=1, keepdims=True)
    one_hot = (keys == idx_f).astype(jnp.float32)

    enc_ref[...] = one_hot
    idx_ref[...] = idx_f.astype(jnp.int32)

    zq_ref[...] = jax.lax.dot_general(one_hot, ewT, (((1,), (1,)), ((), ())),
                                      preferred_element_type=jnp.float32)

    part = jnp.sum(dmin)
    hpart = jnp.sum(one_hot, axis=0)[None, :]

    @pl.when(i == 0)
    def _init():
        loss_ref[...] = jnp.zeros_like(loss_ref)
        hist_ref[...] = jnp.zeros_like(hist_ref)

    loss_ref[...] = loss_ref[...] + part
    hist_ref[...] = hist_ref[...] + hpart

    @pl.when(i == pl.num_programs(0) - 1)
    def _finish():
        loss_ref[...] = (1.0 + _BETA) * loss_ref[...] / _N_TOTAL
        e_mean = hist_ref[...] / (_B * _HW)
        ent = jnp.sum(e_mean * jnp.log(e_mean + 1e-10))
        perp_ref[...] = jnp.exp(-ent) * jnp.ones_like(perp_ref)


def kernel(z, emb_weight):
    zf = jnp.transpose(z, (0, 2, 3, 1)).reshape(_B * _HW, _E_DIM)
    et = emb_weight.T
    zq2, enc, idx, loss, perp = pl.pallas_call(
        _vq_body,
        grid=(_NSTEP,),
        in_specs=[
            pl.BlockSpec((_R, _E_DIM), lambda i: (i, 0)),
            pl.BlockSpec((_E_DIM, _N_E), lambda i: (0, 0)),
        ],
        out_specs=[
            pl.BlockSpec((_R, _E_DIM), lambda i: (i, 0)),
            pl.BlockSpec((_R, _N_E), lambda i: (i, 0)),
            pl.BlockSpec((_R, 1), lambda i: (i, 0)),
            pl.BlockSpec((1, 1), lambda i: (0, 0)),
            pl.BlockSpec((1, 1), lambda i: (0, 0)),
        ],
        out_shape=[
            jax.ShapeDtypeStruct((_B * _HW, _E_DIM), jnp.float32),
            jax.ShapeDtypeStruct((_B * _HW, _N_E), jnp.float32),
            jax.ShapeDtypeStruct((_B * _HW, 1), jnp.int32),
            jax.ShapeDtypeStruct((1, 1), jnp.float32),
            jax.ShapeDtypeStruct((1, 1), jnp.float32),
        ],
        scratch_shapes=[pltpu.VMEM((1, _N_E), jnp.float32)],
    )(zf, et)
    zq = jnp.transpose(zq2.reshape(_B, 32, 32, _E_DIM), (0, 3, 1, 2))
    return (zq, loss.reshape(()), perp.reshape(()), enc, idx)
